# baseline (speedup 1.0000x reference)
import jax
import jax.numpy as jnp
from jax import lax
from jax.experimental import pallas as pl
from jax.experimental.pallas import tpu as pltpu
from jax.experimental.pallas import tpu_sc as plsc

_M = 8
_D = 128
_B = 16384
_C = 10
_MC = _M * _C

_BT = 4096

_NW = 32
_SPT = _B // _NW


def _votes_body(w_ref, b_ref, x_ref, out_ref):
    lm = lax.dot_general(
        w_ref[...], x_ref[...], (((1,), (1,)), ((), ())),
        preferred_element_type=jnp.float32,
    )
    lm = lm + b_ref[...]
    row = lax.broadcasted_iota(jnp.int32, (_MC, _BT), 0)
    cl = row % _C
    val = lm
    idx = cl
    for s in (1, 2, 4, 8):
        sval = pltpu.roll(val, _MC - s, 0)
        sidx = pltpu.roll(idx, _MC - s, 0)
        take = (cl < _C - s) & (sval > val)
        val = jnp.where(take, sval, val)
        idx = jnp.where(take, sidx, idx)
    contrib = jnp.where(cl == 0, idx << ((row // _C) * 4), 0)
    out_ref[...] = jnp.sum(contrib, axis=0, keepdims=True)[None]


def _tc_votes(x, W, b):
    Wp = W.transpose(0, 2, 1).reshape(_MC, _D)
    bp = b.reshape(_MC, 1)
    grid = (_B // _BT,)
    return pl.pallas_call(
        _votes_body,
        grid=grid,
        in_specs=[
            pl.BlockSpec((_MC, _D), lambda i: (0, 0)),
            pl.BlockSpec((_MC, 1), lambda i: (0, 0)),
            pl.BlockSpec((_BT, _D), lambda i: (i, 0)),
        ],
        out_specs=pl.BlockSpec((1, 1, _BT), lambda i: (i, 0, 0)),
        out_shape=jax.ShapeDtypeStruct((_B // _BT, 1, _BT), jnp.int32),
        compiler_params=pltpu.CompilerParams(skip_device_barrier=True),
    )(Wp, bp, x)


def _sc_body(packed_hbm, out_hbm, packed_v, counts_v):
    wid = lax.axis_index("s") * 2 + lax.axis_index("c")
    base = wid * _SPT
    chunks = _BT // _SPT
    pltpu.sync_copy(
        packed_hbm.at[wid // chunks, 0, pl.ds((wid % chunks) * _SPT, _SPT)],
        packed_v)

    lane = lax.iota(jnp.int32, 16)
    zeros = jnp.zeros((16,), jnp.float32)
    ones = jnp.ones((16,), jnp.float32)

    def vote_body(j, carry):
        s = j * 16 + lane
        for c in range(_C):
            cvec = jnp.full((16,), c, jnp.int32)
            plsc.store_scatter(counts_v, [cvec, s], zeros)
        p = packed_v[pl.ds(j * 16, 16)]
        for m in range(_M):
            v = (p >> (4 * m)) & 15
            plsc.addupdate_scatter(counts_v, [v, s], ones)
        return carry

    lax.fori_loop(0, _SPT // 16, vote_body, 0)

    pltpu.sync_copy(counts_v, out_hbm.at[:, pl.ds(base, _SPT)])


def _sc_bincount(packed):
    run = pl.kernel(
        _sc_body,
        mesh=plsc.VectorSubcoreMesh(core_axis_name="c", subcore_axis_name="s"),
        out_type=jax.ShapeDtypeStruct((_C, _B), jnp.float32),
        scratch_types=[
            pltpu.VMEM((_SPT,), jnp.int32),
            pltpu.VMEM((_C, _SPT), jnp.float32),
        ],
        compiler_params=pltpu.CompilerParams(
            needs_layout_passes=False, skip_device_barrier=True),
    )
    return run(packed)


@jax.jit
def kernel(x, W, b):
    packed = _tc_votes(x, W, b)
    return _sc_bincount(packed).T


# speedup vs baseline: 1.1825x; 1.1825x over previous
import jax
import jax.numpy as jnp
from jax import lax
from jax.experimental import pallas as pl
from jax.experimental.pallas import tpu as pltpu
from jax.experimental.pallas import tpu_sc as plsc

_M = 8
_D = 128
_B = 16384
_C = 10
_MC = _M * _C

_BT = 4096

_NW = 32
_SPT = _B // _NW


def _votes_body(w_ref, b_ref, x_ref, out_ref):
    r80 = lax.broadcasted_iota(jnp.int32, (_MC, _M), 0)
    em = (r80 % _M == lax.broadcasted_iota(jnp.int32, (_MC, _M), 1))
    r80c = lax.broadcasted_iota(jnp.int32, (_MC, _C), 0)
    c1 = (r80c // _M == lax.broadcasted_iota(jnp.int32, (_MC, _C), 1))
    brows = lax.dot_general(
        em.astype(jnp.float32), b_ref[...], (((1,), (0,)), ((), ())),
        preferred_element_type=jnp.float32,
    )
    bias = jnp.sum(jnp.where(c1, brows, 0.0), axis=1, keepdims=True)

    lm = lax.dot_general(
        w_ref[...], x_ref[...], (((1,), (1,)), ((), ())),
        preferred_element_type=jnp.float32,
    )
    lm = lm + bias
    row = lax.broadcasted_iota(jnp.int32, (_MC, _BT), 0)
    cl = row >> 3
    val = lm
    idx = cl
    for s in (1, 2, 4, 8):
        sval = pltpu.roll(val, _MC - _M * s, 0)
        sidx = pltpu.roll(idx, _MC - _M * s, 0)
        take = (cl < _C - s) & (sval > val)
        val = jnp.where(take, sval, val)
        idx = jnp.where(take, sidx, idx)
    contrib = jnp.where(cl == 0, idx << ((row & 7) * 4), 0)
    out_ref[...] = jnp.sum(contrib, axis=0, keepdims=True)[None]


def _tc_votes(x, W, b):
    Wp = W.transpose(2, 0, 1).reshape(_MC, _D)
    grid = (_B // _BT,)
    return pl.pallas_call(
        _votes_body,
        grid=grid,
        in_specs=[
            pl.BlockSpec((_MC, _D), lambda i: (0, 0)),
            pl.BlockSpec((_M, _C), lambda i: (0, 0)),
            pl.BlockSpec((_BT, _D), lambda i: (i, 0)),
        ],
        out_specs=pl.BlockSpec((1, 1, _BT), lambda i: (i, 0, 0)),
        out_shape=jax.ShapeDtypeStruct((_B // _BT, 1, _BT), jnp.int32),
        compiler_params=pltpu.CompilerParams(skip_device_barrier=True),
    )(Wp, b, x)


def _sc_body(packed_hbm, out_hbm, packed_v, counts_v):
    wid = lax.axis_index("s") * 2 + lax.axis_index("c")
    base = wid * _SPT
    chunks = _BT // _SPT
    pltpu.sync_copy(
        packed_hbm.at[wid // chunks, 0, pl.ds((wid % chunks) * _SPT, _SPT)],
        packed_v)

    lane = lax.iota(jnp.int32, 16)
    zeros = jnp.zeros((16,), jnp.float32)
    ones = jnp.ones((16,), jnp.float32)

    def vote_body(j, carry):
        s = j * 16 + lane
        for c in range(_C):
            cvec = jnp.full((16,), c, jnp.int32)
            plsc.store_scatter(counts_v, [cvec, s], zeros)
        p = packed_v[pl.ds(j * 16, 16)]
        for m in range(_M):
            v = (p >> (4 * m)) & 15
            plsc.addupdate_scatter(counts_v, [v, s], ones)
        return carry

    lax.fori_loop(0, _SPT // 16, vote_body, 0)

    pltpu.sync_copy(counts_v, out_hbm.at[:, pl.ds(base, _SPT)])


def _sc_bincount(packed):
    run = pl.kernel(
        _sc_body,
        mesh=plsc.VectorSubcoreMesh(core_axis_name="c", subcore_axis_name="s"),
        out_type=jax.ShapeDtypeStruct((_C, _B), jnp.float32),
        scratch_types=[
            pltpu.VMEM((_SPT,), jnp.int32),
            pltpu.VMEM((_C, _SPT), jnp.float32),
        ],
        compiler_params=pltpu.CompilerParams(
            needs_layout_passes=False, skip_device_barrier=True),
    )
    return run(packed)


@jax.jit
def kernel(x, W, b):
    packed = _tc_votes(x, W, b)
    return _sc_bincount(packed).T


# speedup vs baseline: 1.1900x; 1.0064x over previous
import jax
import jax.numpy as jnp
from jax import lax
from jax.experimental import pallas as pl
from jax.experimental.pallas import tpu as pltpu
from jax.experimental.pallas import tpu_sc as plsc

_M = 8
_D = 128
_B = 16384
_C = 10
_MC = _M * _C

_BT = 4096

_NW = 32
_SPT = _B // _NW


def _votes_body(w_ref, b_ref, x_ref, out_ref):
    r80c = lax.broadcasted_iota(jnp.int32, (_MC, _C), 0)
    c1 = (r80c // _M == lax.broadcasted_iota(jnp.int32, (_MC, _C), 1))
    brows = jnp.concatenate([b_ref[...]] * _C, axis=0)
    bias = jnp.sum(jnp.where(c1, brows, 0.0), axis=1, keepdims=True)

    lm = lax.dot_general(
        w_ref[...], x_ref[...], (((1,), (1,)), ((), ())),
        preferred_element_type=jnp.float32,
    )
    lm = lm + bias
    row = lax.broadcasted_iota(jnp.int32, (_MC, _BT), 0)
    cl = row >> 3
    val = lm
    idx = cl
    for s in (1, 2, 4, 8):
        sval = pltpu.roll(val, _MC - _M * s, 0)
        sidx = pltpu.roll(idx, _MC - _M * s, 0)
        take = (cl < _C - s) & (sval > val)
        val = jnp.where(take, sval, val)
        idx = jnp.where(take, sidx, idx)
    contrib = jnp.where(cl == 0, idx << ((row & 7) * 4), 0)
    out_ref[...] = jnp.sum(contrib, axis=0, keepdims=True)[None]


def _tc_votes(x, W, b):
    Wp = W.transpose(2, 0, 1).reshape(_MC, _D)
    grid = (_B // _BT,)
    return pl.pallas_call(
        _votes_body,
        grid=grid,
        in_specs=[
            pl.BlockSpec((_MC, _D), lambda i: (0, 0)),
            pl.BlockSpec((_M, _C), lambda i: (0, 0)),
            pl.BlockSpec((_BT, _D), lambda i: (i, 0)),
        ],
        out_specs=pl.BlockSpec((1, 1, _BT), lambda i: (i, 0, 0)),
        out_shape=jax.ShapeDtypeStruct((_B // _BT, 1, _BT), jnp.int32),
        compiler_params=pltpu.CompilerParams(skip_device_barrier=True),
    )(Wp, b, x)


def _sc_body(packed_hbm, out_hbm, packed_v, counts_v):
    wid = lax.axis_index("s") * 2 + lax.axis_index("c")
    base = wid * _SPT
    chunks = _BT // _SPT
    pltpu.sync_copy(
        packed_hbm.at[wid // chunks, 0, pl.ds((wid % chunks) * _SPT, _SPT)],
        packed_v)

    lane = lax.iota(jnp.int32, 16)
    zeros = jnp.zeros((16,), jnp.float32)
    ones = jnp.ones((16,), jnp.float32)

    def vote_body(j, carry):
        s = j * 16 + lane
        for c in range(_C):
            cvec = jnp.full((16,), c, jnp.int32)
            plsc.store_scatter(counts_v, [cvec, s], zeros)
        p = packed_v[pl.ds(j * 16, 16)]
        for m in range(_M):
            v = (p >> (4 * m)) & 15
            plsc.addupdate_scatter(counts_v, [v, s], ones)
        return carry

    lax.fori_loop(0, _SPT // 16, vote_body, 0)

    pltpu.sync_copy(counts_v, out_hbm.at[:, pl.ds(base, _SPT)])


def _sc_bincount(packed):
    run = pl.kernel(
        _sc_body,
        mesh=plsc.VectorSubcoreMesh(core_axis_name="c", subcore_axis_name="s"),
        out_type=jax.ShapeDtypeStruct((_C, _B), jnp.float32),
        scratch_types=[
            pltpu.VMEM((_SPT,), jnp.int32),
            pltpu.VMEM((_C, _SPT), jnp.float32),
        ],
        compiler_params=pltpu.CompilerParams(
            needs_layout_passes=False, skip_device_barrier=True),
    )
    return run(packed)


@jax.jit
def kernel(x, W, b):
    packed = _tc_votes(x, W, b)
    return _sc_bincount(packed).T
